# Initial kernel scaffold; baseline (speedup 1.0000x reference)
import jax
import jax.numpy as jnp
from jax.experimental import pallas as pl


def kernel(x, edge_index, edge_attr, W1, b1, W2, b2, edge_table, Wc1, bc1, Wc2, bc2, Wc3, bc3, Wh1, bh1, Wh2, bh2):
    raise NotImplementedError("write your pallas kernel here")


# speedup vs baseline: 6.1150x; 6.1150x over previous
import functools

import jax
import jax.numpy as jnp
from jax import lax
from jax.experimental import pallas as pl
from jax.experimental.pallas import tpu as pltpu
from jax.experimental.pallas import tpu_sc as plsc

N_NODES = 10000
N_PAD = 10240
D_IN = 518
D_H = 128
N_EDGES = 320000
NC, NS, LANES = 2, 16, 16
N_TILES = NC * NS
CH = 128
EPT = 10240
CHUNKS = EPT // CH
E_PAD = EPT * N_TILES
ROWS_PER_TILE = N_PAD // NS


def _mesh():
    return plsc.VectorSubcoreMesh(
        core_axis_name="c", subcore_axis_name="s", num_cores=NC, num_subcores=NS)


@functools.partial(
    pl.kernel,
    out_type=jax.ShapeDtypeStruct((NC, N_PAD, LANES), jnp.float32),
    mesh=_mesh(),
    scratch_types=[
        pltpu.VMEM((1, CH), jnp.int32),
        pltpu.VMEM((CH, LANES), jnp.float32),
        pltpu.VMEM_SHARED((N_PAD, LANES), jnp.float32),
    ],
)
def _sc_degree(col_hbm, out_hbm, coli_v, ones_v, cnt_sh):
    cid = lax.axis_index("c")
    sid = lax.axis_index("s")
    wid = cid * NS + sid
    base = wid * CHUNKS

    z16 = jnp.zeros((LANES,), jnp.float32)

    def zero_row(r, carry):
        ones_v[r, :] = z16
        return carry

    lax.fori_loop(0, CH, zero_row, 0)
    for k in range(ROWS_PER_TILE // CH):
        pltpu.sync_copy(ones_v, cnt_sh.at[pl.ds(sid * ROWS_PER_TILE + k * CH, CH)])

    o16 = jnp.ones((LANES,), jnp.float32)

    def ones_row(r, carry):
        ones_v[r, :] = o16
        return carry

    lax.fori_loop(0, CH, ones_row, 0)
    plsc.subcore_barrier()

    def count_body(i, carry):
        pltpu.sync_copy(col_hbm.at[pl.ds(base + i, 1)], coli_v)
        pltpu.sync_copy(ones_v, cnt_sh.at[coli_v.at[0]], add=True)
        return carry

    lax.fori_loop(0, CHUNKS, count_body, 0)

    plsc.subcore_barrier()
    pltpu.sync_copy(cnt_sh.at[pl.ds(sid * ROWS_PER_TILE, ROWS_PER_TILE)],
                    out_hbm.at[cid, pl.ds(sid * ROWS_PER_TILE, ROWS_PER_TILE)])


@functools.partial(
    pl.kernel,
    out_type=jax.ShapeDtypeStruct((NC, N_PAD, D_H), jnp.float32),
    mesh=_mesh(),
    scratch_types=[
        pltpu.VMEM((2, CH), jnp.int32),
        pltpu.VMEM((2, CH), jnp.int32),
        pltpu.VMEM((CH, D_H), jnp.float32),
        pltpu.VMEM((CH, D_H), jnp.float32),
        pltpu.VMEM_SHARED((N_PAD, D_H), jnp.float32),
        pltpu.SemaphoreType.DMA,
        pltpu.SemaphoreType.DMA,
    ],
)
def _sc_conv(row_hbm, col_hbm, hs_hbm, out_hbm,
             rowi_v, coli_v, rows0_v, rows1_v, acc_sh, sem0, sem1):
    cid = lax.axis_index("c")
    sid = lax.axis_index("s")
    wid = cid * NS + sid
    base = wid * CHUNKS

    z16 = jnp.zeros((LANES,), jnp.float32)

    def zero_row(r, _):
        for c8 in range(D_H // LANES):
            rows0_v[r, pl.ds(c8 * LANES, LANES)] = z16
        return _

    lax.fori_loop(0, CH, zero_row, 0)
    for k in range(ROWS_PER_TILE // CH):
        pltpu.sync_copy(rows0_v,
                        acc_sh.at[pl.ds(sid * ROWS_PER_TILE + k * CH, CH)])
    plsc.subcore_barrier()

    rows_bufs = (rows0_v, rows1_v)
    sems = (sem0, sem1)

    def load_and_gather(chunk, buf):
        pltpu.sync_copy(row_hbm.at[pl.ds(base + chunk, 1)], rowi_v.at[pl.ds(buf, 1)])
        pltpu.sync_copy(col_hbm.at[pl.ds(base + chunk, 1)], coli_v.at[pl.ds(buf, 1)])
        return pltpu.async_copy(hs_hbm.at[rowi_v.at[buf]], rows_bufs[buf], sems[buf])

    def scatter(buf):
        pltpu.sync_copy(rows_bufs[buf], acc_sh.at[coli_v.at[buf]], add=True)

    load_and_gather(0, 0).wait()

    def pair_body(p, carry):
        chunk0 = 2 * p
        d1 = load_and_gather(chunk0 + 1, 1)
        scatter(0)
        d1.wait()
        @pl.when(p < CHUNKS // 2 - 1)
        def _not_last():
            d0 = load_and_gather(chunk0 + 2, 0)
            scatter(1)
            d0.wait()

        @pl.when(p == CHUNKS // 2 - 1)
        def _last():
            scatter(1)
        return carry

    lax.fori_loop(0, CHUNKS // 2, pair_body, 0)

    plsc.subcore_barrier()

    pltpu.sync_copy(acc_sh.at[pl.ds(sid * ROWS_PER_TILE, ROWS_PER_TILE)],
                    out_hbm.at[cid, pl.ds(sid * ROWS_PER_TILE, ROWS_PER_TILE)])


_R = 1280
_GRID = N_PAD // _R


def _tc_stage1(xp, W1, b1, W2, b2, Wc1, degp):
    def body(x_ref, w1_ref, b1_ref, w2_ref, b2_ref, wc1_ref, deg_ref, hs_ref):
        h = jnp.maximum(x_ref[...] @ w1_ref[...] + b1_ref[...], 0.0)
        h = h @ w2_ref[...] + b2_ref[...]
        hw = h @ wc1_ref[...]
        deg = (deg_ref[0] + deg_ref[1])[:, :1] + 1.0
        dinv = lax.rsqrt(deg)
        hs_ref[...] = hw * dinv

    return pl.pallas_call(
        body,
        grid=(_GRID,),
        in_specs=[
            pl.BlockSpec((_R, D_IN), lambda i: (i, 0)),
            pl.BlockSpec((D_IN, D_H), lambda i: (0, 0)),
            pl.BlockSpec((1, D_H), lambda i: (0, 0)),
            pl.BlockSpec((D_H, D_H), lambda i: (0, 0)),
            pl.BlockSpec((1, D_H), lambda i: (0, 0)),
            pl.BlockSpec((D_H, D_H), lambda i: (0, 0)),
            pl.BlockSpec((NC, _R, LANES), lambda i: (0, i, 0)),
        ],
        out_specs=pl.BlockSpec((_R, D_H), lambda i: (i, 0)),
        out_shape=jax.ShapeDtypeStruct((N_PAD, D_H), jnp.float32),
    )(xp, W1, b1, W2, b2, Wc1, degp)


def _tc_mid(p, hs_prev, degp, b, Wc_next):
    def body(p_ref, hs_ref, deg_ref, b_ref, wc_ref, out_ref):
        deg = (deg_ref[0] + deg_ref[1])[:, :1] + 1.0
        dinv = lax.rsqrt(deg)
        s = p_ref[0] + p_ref[1] + hs_ref[...]
        h = jnp.maximum(dinv * s + b_ref[...], 0.0)
        out_ref[...] = (h @ wc_ref[...]) * dinv

    return pl.pallas_call(
        body,
        grid=(_GRID,),
        in_specs=[
            pl.BlockSpec((NC, _R, D_H), lambda i: (0, i, 0)),
            pl.BlockSpec((_R, D_H), lambda i: (i, 0)),
            pl.BlockSpec((NC, _R, LANES), lambda i: (0, i, 0)),
            pl.BlockSpec((1, D_H), lambda i: (0, 0)),
            pl.BlockSpec((D_H, D_H), lambda i: (0, 0)),
        ],
        out_specs=pl.BlockSpec((_R, D_H), lambda i: (i, 0)),
        out_shape=jax.ShapeDtypeStruct((N_PAD, D_H), jnp.float32),
    )(p, hs_prev, degp, b, Wc_next)


def _tc_final(p, hs_prev, degp, b, Wh1, bh1, Wh2, bh2):
    def body(p_ref, hs_ref, deg_ref, b_ref, wh1_ref, bh1_ref, wh2_ref,
             bh2_ref, out_ref):
        deg = (deg_ref[0] + deg_ref[1])[:, :1] + 1.0
        dinv = lax.rsqrt(deg)
        h = jnp.maximum(dinv * (p_ref[0] + p_ref[1] + hs_ref[...]) + b_ref[...],
                        0.0)
        ridx = lax.broadcasted_iota(jnp.int32, (N_PAD, D_H), 0)
        h = jnp.where(ridx < N_NODES, h, 0.0)
        g = jnp.sum(h, axis=0, keepdims=True) * (1.0 / N_NODES)
        t = jnp.maximum(g @ wh1_ref[...] + bh1_ref[...], 0.0)
        out_ref[...] = t @ wh2_ref[...] + bh2_ref[...]

    return pl.pallas_call(
        body,
        out_shape=jax.ShapeDtypeStruct((1, bh2.shape[-1]), jnp.float32),
    )(p, hs_prev, degp, b, Wh1, bh1, Wh2, bh2)


def kernel(x, edge_index, edge_attr, W1, b1, W2, b2, edge_table,
           Wc1, bc1, Wc2, bc2, Wc3, bc3, Wh1, bh1, Wh2, bh2):
    del edge_attr, edge_table

    row = edge_index[0]
    col = edge_index[1]
    pad = E_PAD - N_EDGES
    row_p = jnp.concatenate([row, jnp.zeros((pad,), jnp.int32)])
    col_p = jnp.concatenate([col, jnp.full((pad,), N_NODES, jnp.int32)])
    row2d = row_p.reshape(E_PAD // CH, CH)
    col2d = col_p.reshape(E_PAD // CH, CH)

    degp3 = _sc_degree(col2d)

    xp = jnp.pad(x, ((0, N_PAD - N_NODES), (0, 0)))
    b1r = b1.reshape(1, -1)
    b2r = b2.reshape(1, -1)

    hs1 = _tc_stage1(xp, W1, b1r, W2, b2r, Wc1, degp3)
    p1 = _sc_conv(row2d, col2d, hs1)
    hs2 = _tc_mid(p1, hs1, degp3, bc1.reshape(1, -1), Wc2)
    p2 = _sc_conv(row2d, col2d, hs2)
    hs3 = _tc_mid(p2, hs2, degp3, bc2.reshape(1, -1), Wc3)
    p3 = _sc_conv(row2d, col2d, hs3)
    out = _tc_final(p3, hs3, degp3, bc3.reshape(1, -1),
                    Wh1, bh1.reshape(1, -1), Wh2, bh2.reshape(1, -1))
    return out


# speedup vs baseline: 13.6249x; 2.2281x over previous
import functools

import jax
import jax.numpy as jnp
from jax import lax
from jax.experimental import pallas as pl
from jax.experimental.pallas import tpu as pltpu
from jax.experimental.pallas import tpu_sc as plsc

N_NODES = 10000
N_PAD = 10240
D_IN = 518
D_H = 128
N_EDGES = 320000
NC, NS, LANES = 2, 16, 16
N_TILES = NC * NS
CH = 128
EPT = 10240
CHUNKS = EPT // CH
E_PAD = EPT * N_TILES
ROWS_PER_TILE = N_PAD // NS


def _mesh():
    return plsc.VectorSubcoreMesh(
        core_axis_name="c", subcore_axis_name="s", num_cores=NC, num_subcores=NS)


@functools.partial(
    pl.kernel,
    out_type=jax.ShapeDtypeStruct((NC, N_PAD, LANES), jnp.float32),
    mesh=_mesh(),
    scratch_types=[
        pltpu.VMEM((1, CH), jnp.int32),
        pltpu.VMEM((CH, LANES), jnp.float32),
        pltpu.VMEM_SHARED((N_PAD, LANES), jnp.float32),
    ],
)
def _sc_degree(col_hbm, out_hbm, coli_v, ones_v, cnt_sh):
    cid = lax.axis_index("c")
    sid = lax.axis_index("s")
    wid = cid * NS + sid
    base = wid * CHUNKS

    z16 = jnp.zeros((LANES,), jnp.float32)

    def zero_row(r, carry):
        ones_v[r, :] = z16
        return carry

    lax.fori_loop(0, CH, zero_row, 0)
    for k in range(ROWS_PER_TILE // CH):
        pltpu.sync_copy(ones_v, cnt_sh.at[pl.ds(sid * ROWS_PER_TILE + k * CH, CH)])

    o16 = jnp.ones((LANES,), jnp.float32)

    def ones_row(r, carry):
        ones_v[r, :] = o16
        return carry

    lax.fori_loop(0, CH, ones_row, 0)
    plsc.subcore_barrier()

    def count_body(i, carry):
        pltpu.sync_copy(col_hbm.at[pl.ds(base + i, 1)], coli_v)
        pltpu.sync_copy(ones_v, cnt_sh.at[coli_v.at[0]], add=True)
        return carry

    lax.fori_loop(0, CHUNKS, count_body, 0)

    plsc.subcore_barrier()
    pltpu.sync_copy(cnt_sh.at[pl.ds(sid * ROWS_PER_TILE, ROWS_PER_TILE)],
                    out_hbm.at[cid, pl.ds(sid * ROWS_PER_TILE, ROWS_PER_TILE)])


@functools.partial(
    pl.kernel,
    out_type=jax.ShapeDtypeStruct((NC, N_PAD, D_H), jnp.float32),
    mesh=_mesh(),
    scratch_types=[
        pltpu.VMEM((2, CH), jnp.int32),
        pltpu.VMEM((2, CH), jnp.int32),
        pltpu.VMEM((CH, D_H), jnp.float32),
        pltpu.VMEM((CH, D_H), jnp.float32),
        pltpu.VMEM_SHARED((N_PAD, D_H), jnp.float32),
        pltpu.SemaphoreType.DMA,
        pltpu.SemaphoreType.DMA,
    ],
)
def _sc_conv(row_hbm, col_hbm, hs_hbm, out_hbm,
             rowi_v, coli_v, rows0_v, rows1_v, acc_sh, sem0, sem1):
    cid = lax.axis_index("c")
    sid = lax.axis_index("s")
    wid = cid * NS + sid
    base = wid * CHUNKS

    z16 = jnp.zeros((LANES,), jnp.float32)

    def zero_row(r, _):
        for c8 in range(D_H // LANES):
            rows0_v[r, pl.ds(c8 * LANES, LANES)] = z16
        return _

    lax.fori_loop(0, CH, zero_row, 0)
    for k in range(ROWS_PER_TILE // CH):
        pltpu.sync_copy(rows0_v,
                        acc_sh.at[pl.ds(sid * ROWS_PER_TILE + k * CH, CH)])
    plsc.subcore_barrier()

    rows_bufs = (rows0_v, rows1_v)
    sems = (sem0, sem1)

    def load_and_gather(chunk, buf):
        pltpu.sync_copy(row_hbm.at[pl.ds(base + chunk, 1)], rowi_v.at[pl.ds(buf, 1)])
        pltpu.sync_copy(col_hbm.at[pl.ds(base + chunk, 1)], coli_v.at[pl.ds(buf, 1)])
        return pltpu.async_copy(hs_hbm.at[rowi_v.at[buf]], rows_bufs[buf], sems[buf])

    def scatter(buf):
        pltpu.sync_copy(rows_bufs[buf], acc_sh.at[coli_v.at[buf]], add=True)

    load_and_gather(0, 0).wait()

    def pair_body(p, carry):
        chunk0 = 2 * p
        d1 = load_and_gather(chunk0 + 1, 1)
        scatter(0)
        d1.wait()
        @pl.when(p < CHUNKS // 2 - 1)
        def _not_last():
            d0 = load_and_gather(chunk0 + 2, 0)
            scatter(1)
            d0.wait()

        @pl.when(p == CHUNKS // 2 - 1)
        def _last():
            scatter(1)
        return carry

    lax.fori_loop(0, CHUNKS // 2, pair_body, 0)

    plsc.subcore_barrier()

    pltpu.sync_copy(acc_sh.at[pl.ds(sid * ROWS_PER_TILE, ROWS_PER_TILE)],
                    out_hbm.at[cid, pl.ds(sid * ROWS_PER_TILE, ROWS_PER_TILE)])


_R = 1280
_GRID = N_PAD // _R


def _tc_stage1(xp, W1, b1, W2, b2, Wc1, degp):
    def body(x_ref, w1_ref, b1_ref, w2_ref, b2_ref, wc1_ref, deg_ref, hs_ref):
        h = jnp.maximum(x_ref[...] @ w1_ref[...] + b1_ref[...], 0.0)
        h = h @ w2_ref[...] + b2_ref[...]
        hw = h @ wc1_ref[...]
        deg = (deg_ref[0] + deg_ref[1])[:, :1] + 1.0
        dinv = lax.rsqrt(deg)
        hs_ref[...] = hw * dinv

    return pl.pallas_call(
        body,
        grid=(_GRID,),
        in_specs=[
            pl.BlockSpec((_R, D_IN), lambda i: (i, 0)),
            pl.BlockSpec((D_IN, D_H), lambda i: (0, 0)),
            pl.BlockSpec((1, D_H), lambda i: (0, 0)),
            pl.BlockSpec((D_H, D_H), lambda i: (0, 0)),
            pl.BlockSpec((1, D_H), lambda i: (0, 0)),
            pl.BlockSpec((D_H, D_H), lambda i: (0, 0)),
            pl.BlockSpec((NC, _R, LANES), lambda i: (0, i, 0)),
        ],
        out_specs=pl.BlockSpec((_R, D_H), lambda i: (i, 0)),
        out_shape=jax.ShapeDtypeStruct((N_PAD, D_H), jnp.float32),
    )(xp, W1, b1, W2, b2, Wc1, degp)


def _tc_mid(p, hs_prev, degp, b, Wc_next):
    def body(p_ref, hs_ref, deg_ref, b_ref, wc_ref, out_ref):
        deg = (deg_ref[0] + deg_ref[1])[:, :1] + 1.0
        dinv = lax.rsqrt(deg)
        s = p_ref[0] + p_ref[1] + hs_ref[...]
        h = jnp.maximum(dinv * s + b_ref[...], 0.0)
        out_ref[...] = (h @ wc_ref[...]) * dinv

    return pl.pallas_call(
        body,
        grid=(_GRID,),
        in_specs=[
            pl.BlockSpec((NC, _R, D_H), lambda i: (0, i, 0)),
            pl.BlockSpec((_R, D_H), lambda i: (i, 0)),
            pl.BlockSpec((NC, _R, LANES), lambda i: (0, i, 0)),
            pl.BlockSpec((1, D_H), lambda i: (0, 0)),
            pl.BlockSpec((D_H, D_H), lambda i: (0, 0)),
        ],
        out_specs=pl.BlockSpec((_R, D_H), lambda i: (i, 0)),
        out_shape=jax.ShapeDtypeStruct((N_PAD, D_H), jnp.float32),
    )(p, hs_prev, degp, b, Wc_next)


def _tc_final(p, hs_prev, degp, b, Wh1, bh1, Wh2, bh2):
    def body(p_ref, hs_ref, deg_ref, b_ref, wh1_ref, bh1_ref, wh2_ref,
             bh2_ref, out_ref):
        deg = (deg_ref[0] + deg_ref[1])[:, :1] + 1.0
        dinv = lax.rsqrt(deg)
        h = jnp.maximum(dinv * (p_ref[0] + p_ref[1] + hs_ref[...]) + b_ref[...],
                        0.0)
        ridx = lax.broadcasted_iota(jnp.int32, (N_PAD, D_H), 0)
        h = jnp.where(ridx < N_NODES, h, 0.0)
        g = jnp.sum(h, axis=0, keepdims=True) * (1.0 / N_NODES)
        t = jnp.maximum(g @ wh1_ref[...] + bh1_ref[...], 0.0)
        out_ref[...] = t @ wh2_ref[...] + bh2_ref[...]

    return pl.pallas_call(
        body,
        out_shape=jax.ShapeDtypeStruct((1, bh2.shape[-1]), jnp.float32),
    )(p, hs_prev, degp, b, Wh1, bh1, Wh2, bh2)


def kernel(x, edge_index, edge_attr, W1, b1, W2, b2, edge_table,
           Wc1, bc1, Wc2, bc2, Wc3, bc3, Wh1, bh1, Wh2, bh2):
    del edge_attr, edge_table

    row = edge_index[0]
    col = edge_index[1]
    pad = E_PAD - N_EDGES
    pad_i = jnp.arange(pad, dtype=jnp.int32)
    row_p = jnp.concatenate([row, (pad_i * 37) % N_NODES])
    col_p = jnp.concatenate([col, N_NODES + pad_i % (N_PAD - N_NODES)])
    row2d = row_p.reshape(E_PAD // CH, CH)
    col2d = col_p.reshape(E_PAD // CH, CH)

    degp3 = _sc_degree(col2d)

    xp = jnp.pad(x, ((0, N_PAD - N_NODES), (0, 0)))
    b1r = b1.reshape(1, -1)
    b2r = b2.reshape(1, -1)

    hs1 = _tc_stage1(xp, W1, b1r, W2, b2r, Wc1, degp3)
    p1 = _sc_conv(row2d, col2d, hs1)
    hs2 = _tc_mid(p1, hs1, degp3, bc1.reshape(1, -1), Wc2)
    p2 = _sc_conv(row2d, col2d, hs2)
    hs3 = _tc_mid(p2, hs2, degp3, bc2.reshape(1, -1), Wc3)
    p3 = _sc_conv(row2d, col2d, hs3)
    out = _tc_final(p3, hs3, degp3, bc3.reshape(1, -1),
                    Wh1, bh1.reshape(1, -1), Wh2, bh2.reshape(1, -1))
    return out


# speedup vs baseline: 22.7894x; 1.6726x over previous
import functools

import jax
import jax.numpy as jnp
from jax import lax
from jax.experimental import pallas as pl
from jax.experimental.pallas import tpu as pltpu
from jax.experimental.pallas import tpu_sc as plsc

N_NODES = 10000
N_PAD = 10240
D_IN = 518
D_H = 128
N_EDGES = 320000
NC, NS, LANES = 2, 16, 16
N_TILES = NC * NS
CH = 128
EPT = 10240
CHUNKS = EPT // CH
E_PAD = EPT * N_TILES
ROWS_PER_TILE = N_PAD // NS


def _mesh():
    return plsc.VectorSubcoreMesh(
        core_axis_name="c", subcore_axis_name="s", num_cores=NC, num_subcores=NS)


@functools.partial(
    pl.kernel,
    out_type=jax.ShapeDtypeStruct((NC, N_PAD, LANES), jnp.float32),
    mesh=_mesh(),
    scratch_types=[
        pltpu.VMEM((CHUNKS, CH), jnp.int32),
        pltpu.VMEM((CH, LANES), jnp.float32),
        pltpu.VMEM_SHARED((N_PAD, LANES), jnp.float32),
    ],
)
def _sc_degree(col_hbm, out_hbm, coli_v, ones_v, cnt_sh):
    cid = lax.axis_index("c")
    sid = lax.axis_index("s")
    wid = cid * NS + sid
    base = wid * CHUNKS

    pltpu.sync_copy(col_hbm.at[pl.ds(base, CHUNKS)], coli_v)

    z16 = jnp.zeros((LANES,), jnp.float32)

    def zero_row(r, carry):
        ones_v[r, :] = z16
        return carry

    lax.fori_loop(0, CH, zero_row, 0)
    for k in range(ROWS_PER_TILE // CH):
        pltpu.sync_copy(ones_v, cnt_sh.at[pl.ds(sid * ROWS_PER_TILE + k * CH, CH)])

    o16 = jnp.ones((LANES,), jnp.float32)

    def ones_row(r, carry):
        ones_v[r, :] = o16
        return carry

    lax.fori_loop(0, CH, ones_row, 0)
    plsc.subcore_barrier()

    def count_body(i, carry):
        pltpu.sync_copy(ones_v, cnt_sh.at[coli_v.at[i]], add=True)
        return carry

    lax.fori_loop(0, CHUNKS, count_body, 0)

    plsc.subcore_barrier()
    pltpu.sync_copy(cnt_sh.at[pl.ds(sid * ROWS_PER_TILE, ROWS_PER_TILE)],
                    out_hbm.at[cid, pl.ds(sid * ROWS_PER_TILE, ROWS_PER_TILE)])


@functools.partial(
    pl.kernel,
    out_type=jax.ShapeDtypeStruct((NC, N_PAD, D_H), jnp.float32),
    mesh=_mesh(),
    scratch_types=[
        pltpu.VMEM((CHUNKS // 2, CH), jnp.int32),
        pltpu.VMEM((CHUNKS // 2, CH), jnp.int32),
        pltpu.VMEM((CH, D_H), jnp.float32),
        pltpu.VMEM((CH, D_H), jnp.float32),
        pltpu.VMEM_SHARED((N_PAD, D_H), jnp.float32),
        pltpu.SemaphoreType.DMA,
        pltpu.SemaphoreType.DMA,
    ],
)
def _sc_conv(row_hbm, col_hbm, hs_hbm, out_hbm,
             rowi_v, coli_v, rows0_v, rows1_v, acc_sh, sem0, sem1):
    cid = lax.axis_index("c")
    sid = lax.axis_index("s")
    wid = cid * NS + sid
    base = wid * CHUNKS
    HALF = CHUNKS // 2

    z16 = jnp.zeros((LANES,), jnp.float32)

    def zero_row(r, _):
        for c8 in range(D_H // LANES):
            rows0_v[r, pl.ds(c8 * LANES, LANES)] = z16
        return _

    lax.fori_loop(0, CH, zero_row, 0)
    for k in range(ROWS_PER_TILE // CH):
        pltpu.sync_copy(rows0_v,
                        acc_sh.at[pl.ds(sid * ROWS_PER_TILE + k * CH, CH)])
    plsc.subcore_barrier()

    rows_bufs = (rows0_v, rows1_v)
    sems = (sem0, sem1)

    def gather(lchunk, buf):
        return pltpu.async_copy(hs_hbm.at[rowi_v.at[lchunk]], rows_bufs[buf],
                                sems[buf])

    def scatter(lchunk, buf):
        pltpu.sync_copy(rows_bufs[buf], acc_sh.at[coli_v.at[lchunk]], add=True)

    for h in range(2):
        pltpu.sync_copy(row_hbm.at[pl.ds(base + h * HALF, HALF)], rowi_v)
        pltpu.sync_copy(col_hbm.at[pl.ds(base + h * HALF, HALF)], coli_v)
        gather(0, 0).wait()

        def pair_body(p, carry):
            chunk0 = 2 * p
            d1 = gather(chunk0 + 1, 1)
            scatter(chunk0, 0)
            d1.wait()
            @pl.when(p < HALF // 2 - 1)
            def _not_last():
                d0 = gather(chunk0 + 2, 0)
                scatter(chunk0 + 1, 1)
                d0.wait()

            @pl.when(p == HALF // 2 - 1)
            def _last():
                scatter(chunk0 + 1, 1)
            return carry

        lax.fori_loop(0, HALF // 2, pair_body, 0)

    plsc.subcore_barrier()

    pltpu.sync_copy(acc_sh.at[pl.ds(sid * ROWS_PER_TILE, ROWS_PER_TILE)],
                    out_hbm.at[cid, pl.ds(sid * ROWS_PER_TILE, ROWS_PER_TILE)])


_R = 1280
_GRID = N_PAD // _R


def _tc_stage1(x, W1, b1, W2, b2, Wc1, degp):
    def body(x_ref, w1_ref, b1_ref, w2_ref, b2_ref, wc1_ref, deg_ref, hs_ref):
        h = jnp.maximum(x_ref[...] @ w1_ref[...] + b1_ref[...], 0.0)
        h = h @ w2_ref[...] + b2_ref[...]
        hw = h @ wc1_ref[...]
        deg = (deg_ref[0] + deg_ref[1])[:, :1] + 1.0
        dinv = lax.rsqrt(deg)
        hs_ref[...] = hw * dinv

    return pl.pallas_call(
        body,
        grid=(_GRID,),
        in_specs=[
            pl.BlockSpec((_R, D_IN), lambda i: (i, 0)),
            pl.BlockSpec((D_IN, D_H), lambda i: (0, 0)),
            pl.BlockSpec((1, D_H), lambda i: (0, 0)),
            pl.BlockSpec((D_H, D_H), lambda i: (0, 0)),
            pl.BlockSpec((1, D_H), lambda i: (0, 0)),
            pl.BlockSpec((D_H, D_H), lambda i: (0, 0)),
            pl.BlockSpec((NC, _R, LANES), lambda i: (0, i, 0)),
        ],
        out_specs=pl.BlockSpec((_R, D_H), lambda i: (i, 0)),
        out_shape=jax.ShapeDtypeStruct((N_PAD, D_H), jnp.float32),
    )(x, W1, b1, W2, b2, Wc1, degp)


def _tc_mid(p, hs_prev, degp, b, Wc_next):
    def body(p_ref, hs_ref, deg_ref, b_ref, wc_ref, out_ref):
        deg = (deg_ref[0] + deg_ref[1])[:, :1] + 1.0
        dinv = lax.rsqrt(deg)
        s = p_ref[0] + p_ref[1] + hs_ref[...]
        h = jnp.maximum(dinv * s + b_ref[...], 0.0)
        out_ref[...] = (h @ wc_ref[...]) * dinv

    return pl.pallas_call(
        body,
        grid=(_GRID,),
        in_specs=[
            pl.BlockSpec((NC, _R, D_H), lambda i: (0, i, 0)),
            pl.BlockSpec((_R, D_H), lambda i: (i, 0)),
            pl.BlockSpec((NC, _R, LANES), lambda i: (0, i, 0)),
            pl.BlockSpec((1, D_H), lambda i: (0, 0)),
            pl.BlockSpec((D_H, D_H), lambda i: (0, 0)),
        ],
        out_specs=pl.BlockSpec((_R, D_H), lambda i: (i, 0)),
        out_shape=jax.ShapeDtypeStruct((N_PAD, D_H), jnp.float32),
    )(p, hs_prev, degp, b, Wc_next)


def _tc_final(p, hs_prev, degp, b, Wh1, bh1, Wh2, bh2):
    def body(p_ref, hs_ref, deg_ref, b_ref, wh1_ref, bh1_ref, wh2_ref,
             bh2_ref, out_ref):
        deg = (deg_ref[0] + deg_ref[1])[:, :1] + 1.0
        dinv = lax.rsqrt(deg)
        h = jnp.maximum(dinv * (p_ref[0] + p_ref[1] + hs_ref[...]) + b_ref[...],
                        0.0)
        ridx = lax.broadcasted_iota(jnp.int32, (N_PAD, D_H), 0)
        h = jnp.where(ridx < N_NODES, h, 0.0)
        g = jnp.sum(h, axis=0, keepdims=True) * (1.0 / N_NODES)
        t = jnp.maximum(g @ wh1_ref[...] + bh1_ref[...], 0.0)
        out_ref[...] = t @ wh2_ref[...] + bh2_ref[...]

    return pl.pallas_call(
        body,
        out_shape=jax.ShapeDtypeStruct((1, bh2.shape[-1]), jnp.float32),
    )(p, hs_prev, degp, b, Wh1, bh1, Wh2, bh2)


def kernel(x, edge_index, edge_attr, W1, b1, W2, b2, edge_table,
           Wc1, bc1, Wc2, bc2, Wc3, bc3, Wh1, bh1, Wh2, bh2):
    del edge_attr, edge_table

    row = edge_index[0]
    col = edge_index[1]
    pad = E_PAD - N_EDGES
    pad_i = jnp.arange(pad, dtype=jnp.int32)
    row_p = jnp.concatenate([row, (pad_i * 37) % N_NODES])
    col_p = jnp.concatenate([col, N_NODES + pad_i % (N_PAD - N_NODES)])
    row2d = row_p.reshape(E_PAD // CH, CH)
    col2d = col_p.reshape(E_PAD // CH, CH)

    degp3 = _sc_degree(col2d)

    b1r = b1.reshape(1, -1)
    b2r = b2.reshape(1, -1)

    hs1 = _tc_stage1(x, W1, b1r, W2, b2r, Wc1, degp3)
    p1 = _sc_conv(row2d, col2d, hs1)
    hs2 = _tc_mid(p1, hs1, degp3, bc1.reshape(1, -1), Wc2)
    p2 = _sc_conv(row2d, col2d, hs2)
    hs3 = _tc_mid(p2, hs2, degp3, bc2.reshape(1, -1), Wc3)
    p3 = _sc_conv(row2d, col2d, hs3)
    out = _tc_final(p3, hs3, degp3, bc3.reshape(1, -1),
                    Wh1, bh1.reshape(1, -1), Wh2, bh2.reshape(1, -1))
    return out
